# baseline (speedup 1.0000x reference)
import dataclasses

import jax
import jax.numpy as jnp
import numpy as np
from jax import lax
from jax.experimental import pallas as pl
from jax.experimental.pallas import tpu as pltpu
from jax.experimental.pallas import tpu_sc as plsc

B = 4096
V = 100000
D = 64
F = 26
NF = F + 1
NUM_IDX = B * F
NW = 32
IDX_PER_W = NUM_IDX // NW
BS = 512
_NL = 16
_WIN = 128


def _sc_gather(flat_tables, flat_idx, pos, lane_ids):
    mesh = plsc.VectorSubcoreMesh(core_axis_name="core", subcore_axis_name="subcore")
    cp = pltpu.CompilerParams()
    if "needs_layout_passes" in pltpu.CompilerParams.__dataclass_fields__:
        cp = dataclasses.replace(cp, needs_layout_passes=False)

    @pl.kernel(
        out_type=jax.ShapeDtypeStruct((NUM_IDX, D), flat_tables.dtype),
        mesh=mesh,
        compiler_params=cp,
        scratch_types=[
            pltpu.VMEM((_NL,), jnp.int32),
            pltpu.SemaphoreType.DMA,
        ],
    )
    def gather_kernel(x_hbm, i_hbm, p_hbm, l_hbm, o_hbm, lane_v, sem):
        pltpu.sync_copy(l_hbm.at[0], lane_v)
        lanes = lane_v[...]

        def body(i_vmem, p_vmem):
            for c in range(_WIN // _NL):
                idx16 = i_vmem[0, pl.ds(c * _NL, _NL)]
                pos16 = p_vmem[0, pl.ds(c * _NL, _NL)]
                for l in range(_NL):
                    m = lanes == l
                    idx = jnp.max(jnp.where(m, idx16, 0))
                    dst = jnp.max(jnp.where(m, pos16, 0))
                    pltpu.async_copy(
                        x_hbm.at[pl.ds(idx, 1)],
                        o_hbm.at[pl.ds(dst, 1)],
                        sem,
                    )

        pltpu.emit_pipeline(
            body,
            grid=(NUM_IDX // _WIN,),
            in_specs=[
                pl.BlockSpec((1, _WIN), lambda i: (0, i)),
                pl.BlockSpec((1, _WIN), lambda i: (0, i)),
            ],
            out_specs=[],
            core_axis_name=("core", "subcore"),
            dimension_semantics=(pltpu.PARALLEL,),
        )(i_hbm, p_hbm)

        @pl.loop(0, IDX_PER_W)
        def _(j):
            pltpu.make_async_copy(
                x_hbm.at[pl.ds(0, 1)],
                o_hbm.at[pl.ds(0, 1)],
                sem,
            ).wait()

    return gather_kernel(flat_tables, flat_idx, pos, lane_ids)


def _dense_body(x_ref, emb_ref, dw1_, db1_, dw2_, db2_, dw3_, db3_,
                ow1d_, ow1z_, ob1_, ow2_, ob2_, ow3_, ob3_, o_ref):
    f32 = jnp.float32
    x = x_ref[...]
    d = jnp.maximum(jax.lax.dot(x, dw1_[...], preferred_element_type=f32) + db1_[...], 0.0)
    d = jnp.maximum(jax.lax.dot(d, dw2_[...], preferred_element_type=f32) + db2_[...], 0.0)
    d = jnp.maximum(jax.lax.dot(d, dw3_[...], preferred_element_type=f32) + db3_[...], 0.0)
    emb = emb_ref[...]
    a = jnp.concatenate([d[None], emb], axis=0)
    z = jax.lax.dot_general(
        a, a, (((2,), (2,)), ((1,), (1,))), preferred_element_type=f32
    )
    zf = z.reshape(BS, NF * NF)
    h = (jax.lax.dot(d, ow1d_[...], preferred_element_type=f32)
         + jax.lax.dot(zf, ow1z_[...], preferred_element_type=f32)
         + ob1_[...])
    h = jnp.maximum(h, 0.0)
    h = jnp.maximum(jax.lax.dot(h, ow2_[...], preferred_element_type=f32) + ob2_[...], 0.0)
    o_ref[...] = jax.lax.dot(h, ow3_[...], preferred_element_type=f32) + ob3_[...]


_LI, _LJ = np.triu_indices(NF, k=1)
_PAIR_POS = np.asarray(_LI * NF + _LJ)


def kernel(dense_features, sparse_indices, tables, dw1, db1, dw2, db2, dw3,
           db3, ow1, ob1, ow2, ob2, ow3, ob3):
    flat_tables = tables.reshape(F * V, D)
    offs = (jnp.arange(F, dtype=jnp.int32) * V)[:, None]
    flat_idx = (sparse_indices.astype(jnp.int32).T + offs).reshape(1, NUM_IDX)
    pos = jnp.arange(NUM_IDX, dtype=jnp.int32)[None, :]
    lane_ids = jnp.arange(_NL, dtype=jnp.int32)[None, :]
    emb3 = _sc_gather(flat_tables, flat_idx, pos, lane_ids).reshape(F, B, D)

    ow1d = ow1[:D]
    ow1z = jnp.zeros((NF * NF, ow1.shape[1]), ow1.dtype).at[_PAIR_POS].set(ow1[D:])

    n_blocks = B // BS
    wspec = lambda shape: pl.BlockSpec(shape, lambda i: (0,) * len(shape))
    out = pl.pallas_call(
        _dense_body,
        grid=(n_blocks,),
        in_specs=[
            pl.BlockSpec((BS, dense_features.shape[1]), lambda i: (i, 0)),
            pl.BlockSpec((F, BS, D), lambda i: (0, i, 0)),
            wspec(dw1.shape), wspec((1, db1.shape[0])),
            wspec(dw2.shape), wspec((1, db2.shape[0])),
            wspec(dw3.shape), wspec((1, db3.shape[0])),
            wspec(ow1d.shape), wspec(ow1z.shape), wspec((1, ob1.shape[0])),
            wspec(ow2.shape), wspec((1, ob2.shape[0])),
            wspec(ow3.shape), wspec((1, ob3.shape[0])),
        ],
        out_specs=pl.BlockSpec((BS, 1), lambda i: (i, 0)),
        out_shape=jax.ShapeDtypeStruct((B, 1), jnp.float32),
    )(
        dense_features, emb3, dw1, db1[None], dw2, db2[None], dw3,
        db3[None], ow1d, ow1z, ob1[None], ow2, ob2[None], ow3, ob3[None],
    )
    return out


# speedup vs baseline: 1.3632x; 1.3632x over previous
import jax
import jax.numpy as jnp
import numpy as np
from jax.experimental import pallas as pl
from jax.experimental.pallas import tpu as pltpu
from jax.experimental.pallas import tpu_sc as plsc

B = 4096
V = 100000
D = 64
F = 26
NF = F + 1
NUM_IDX = B * F
BS = 512
_WIN = 128
_RB = 10000


def _relayout_body(x1_ref, x2_ref, o_ref):
    o_ref[...] = jnp.concatenate([x1_ref[0], x2_ref[0]], axis=1)


def _pair_relayout(tables):
    grid = (F // 2, V // _RB)
    return pl.pallas_call(
        _relayout_body,
        grid=grid,
        in_specs=[
            pl.BlockSpec((1, _RB, D), lambda f, v: (f, v, 0)),
            pl.BlockSpec((1, _RB, D), lambda f, v: (f + F // 2, v, 0)),
        ],
        out_specs=pl.BlockSpec((_RB, 2 * D), lambda f, v: (f * (V // _RB) + v, 0)),
        out_shape=jax.ShapeDtypeStruct((F * V // 2, 2 * D), tables.dtype),
    )(tables, tables)


def _sc_gather(pair_tables, pair_idx):
    mesh = plsc.VectorSubcoreMesh(core_axis_name="core", subcore_axis_name="subcore")

    @pl.kernel(
        out_type=jax.ShapeDtypeStruct((NUM_IDX, 2 * D), pair_tables.dtype),
        mesh=mesh,
    )
    def gather_kernel(x_hbm, i_hbm, o_hbm):
        def body(i_vmem, o_vmem):
            pltpu.sync_copy(x_hbm.at[i_vmem.at[0]], o_vmem)

        pltpu.emit_pipeline(
            body,
            grid=(NUM_IDX // _WIN,),
            in_specs=[pl.BlockSpec((1, _WIN), lambda i: (0, i))],
            out_specs=[pl.BlockSpec((_WIN, 2 * D), lambda i: (i, 0))],
            core_axis_name=("core", "subcore"),
            dimension_semantics=(pltpu.PARALLEL,),
        )(i_hbm, o_hbm)

    return gather_kernel(pair_tables, pair_idx)


def _dense_body(x_ref, emb_ref, dw1_, db1_, dw2_, db2_, dw3_, db3_,
                ow1d_, ow1z_, ob1_, ow2_, ob2_, ow3_, ob3_, o_ref):
    f32 = jnp.float32
    x = x_ref[...]
    d = jnp.maximum(jax.lax.dot(x, dw1_[...], preferred_element_type=f32) + db1_[...], 0.0)
    d = jnp.maximum(jax.lax.dot(d, dw2_[...], preferred_element_type=f32) + db2_[...], 0.0)
    d = jnp.maximum(jax.lax.dot(d, dw3_[...], preferred_element_type=f32) + db3_[...], 0.0)
    g = emb_ref[...]
    emb = jnp.concatenate([g[: F // 2, :, :D], g[F // 2 :, :, D:]], axis=0)
    a = jnp.concatenate([d[None], emb], axis=0)
    z = jax.lax.dot_general(
        a, a, (((2,), (2,)), ((1,), (1,))), preferred_element_type=f32
    )
    zf = z.reshape(BS, NF * NF)
    h = (jax.lax.dot(d, ow1d_[...], preferred_element_type=f32)
         + jax.lax.dot(zf, ow1z_[...], preferred_element_type=f32)
         + ob1_[...])
    h = jnp.maximum(h, 0.0)
    h = jnp.maximum(jax.lax.dot(h, ow2_[...], preferred_element_type=f32) + ob2_[...], 0.0)
    o_ref[...] = jax.lax.dot(h, ow3_[...], preferred_element_type=f32) + ob3_[...]


_LI, _LJ = np.triu_indices(NF, k=1)
_PAIR_POS = np.asarray(_LI * NF + _LJ)


def kernel(dense_features, sparse_indices, tables, dw1, db1, dw2, db2, dw3,
           db3, ow1, ob1, ow2, ob2, ow3, ob3):
    pair_tables = _pair_relayout(tables)

    offs = ((jnp.arange(F, dtype=jnp.int32) % (F // 2)) * V)[:, None]
    pair_idx = (sparse_indices.astype(jnp.int32).T + offs).reshape(1, NUM_IDX)

    emb3 = _sc_gather(pair_tables, pair_idx).reshape(F, B, 2 * D)

    ow1d = ow1[:D]
    ow1z = jnp.zeros((NF * NF, ow1.shape[1]), ow1.dtype).at[_PAIR_POS].set(ow1[D:])

    n_blocks = B // BS
    wspec = lambda shape: pl.BlockSpec(shape, lambda i: (0,) * len(shape))
    out = pl.pallas_call(
        _dense_body,
        grid=(n_blocks,),
        in_specs=[
            pl.BlockSpec((BS, dense_features.shape[1]), lambda i: (i, 0)),
            pl.BlockSpec((F, BS, 2 * D), lambda i: (0, i, 0)),
            wspec(dw1.shape), wspec((1, db1.shape[0])),
            wspec(dw2.shape), wspec((1, db2.shape[0])),
            wspec(dw3.shape), wspec((1, db3.shape[0])),
            wspec(ow1d.shape), wspec(ow1z.shape), wspec((1, ob1.shape[0])),
            wspec(ow2.shape), wspec((1, ob2.shape[0])),
            wspec(ow3.shape), wspec((1, ob3.shape[0])),
        ],
        out_specs=pl.BlockSpec((BS, 1), lambda i: (i, 0)),
        out_shape=jax.ShapeDtypeStruct((B, 1), jnp.float32),
    )(
        dense_features, emb3, dw1, db1[None], dw2, db2[None], dw3,
        db3[None], ow1d, ow1z, ob1[None], ow2, ob2[None], ow3, ob3[None],
    )
    return out
